# Initial kernel scaffold; baseline (speedup 1.0000x reference)
import jax
import jax.numpy as jnp
from jax.experimental import pallas as pl


def kernel(start_index, goal_index, cost_maps, nodes, adj, weighted_adj):
    raise NotImplementedError("write your pallas kernel here")


# speedup vs baseline: 20.8092x; 20.8092x over previous
import dataclasses
import functools
import math

import jax
import jax.numpy as jnp
from jax import lax
from jax.experimental import pallas as pl
from jax.experimental.pallas import tpu as pltpu
from jax.experimental.pallas import tpu_sc as plsc

N = 4096
NS = 16
L = 16
CHUNK = N // NS
NCH = CHUNK // L
TMAX = int(0.05 * N)
INV_SQRT = 1.0 / math.sqrt(N)
INF = float("inf")


def _lane_iota():
    return lax.iota(jnp.int32, L)


def _astar_body(cost_hbm, adj_hbm, wadj_hbm, sg_hbm, hist_out, path_out,
                h_v, g_v, open_v, hist_v, par_v, arow_v, wrow_v, sg_v,
                row_v, cand_v, parall_v, path_v, cand_sh, par_sh, sem):
    cid = lax.axis_index("c")
    sid = lax.axis_index("s")

    @pl.when(cid == 0)
    def _work():
        base = sid * CHUNK
        lane = _lane_iota()

        pltpu.async_copy(sg_hbm, sg_v, sem).wait()
        sgv = sg_v[...]
        start = sgv[0]
        goal = sgv[1]
        goal_f = goal.astype(jnp.float32)

        pltpu.async_copy(cost_hbm.at[sid], h_v, sem).wait()
        pltpu.async_copy(wadj_hbm.at[start, sid], g_v, sem).wait()

        def init_body(c, _):
            sl = pl.ds(c * L, L)
            gi = base + c * L + lane
            gv = g_v[sl]
            gv = jnp.where(gv == INF, 0.0, gv)
            gv = jnp.where(gi == start, 0.0, gv)
            g_v[sl] = gv
            open_v[sl] = jnp.where(gi == start, 1.0, 0.0)
            hist_v[sl] = jnp.zeros((L,), jnp.float32)
            par_v[sl] = lax.broadcast(goal_f, (L,))
            return 0

        lax.fori_loop(0, NCH, init_body, 0, unroll=False)

        def step(t, _):
            def amax_body(c, carry):
                best, bidx = carry
                sl = pl.ds(c * L, L)
                fe = jnp.exp((g_v[sl] * 0.5 + h_v[sl] * 0.5) * (-INV_SQRT))
                fe = fe * open_v[sl]
                m = jnp.max(fe)
                first = jnp.max(plsc.all_reduce_ffs(fe == m))
                take = m > best
                best = jnp.where(take, m, best)
                bidx = jnp.where(take, c * L + first, bidx)
                return best, bidx

            best, bidx = lax.fori_loop(
                0, NCH, amax_body, (jnp.float32(-1.0), jnp.int32(0)),
                unroll=False)
            bv = g_v[pl.ds((bidx >> 4) << 4, L)]
            gbest = jnp.sum(jnp.where(lane == (bidx & 15), bv, 0.0))

            row = jnp.where(lane == 0, best, 0.0)
            row = jnp.where(lane == 1, (base + bidx).astype(jnp.float32), row)
            row = jnp.where(lane == 2, gbest, row)
            row_v[...] = row
            pltpu.sync_copy(row_v, cand_sh.at[sid])
            plsc.subcore_barrier()
            pltpu.sync_copy(cand_sh, cand_v)
            plsc.subcore_barrier()

            vals = plsc.load_gather(cand_v, [lane, jnp.zeros((L,), jnp.int32)])
            gm = jnp.max(vals)
            tstar = jnp.max(plsc.all_reduce_ffs(vals == gm))
            crow = cand_v[tstar, pl.ds(0, L)]
            ind = crow[1].astype(jnp.int32)
            gsel = crow[2]
            ind_f = ind.astype(jnp.float32)

            acpy = pltpu.async_copy(adj_hbm.at[ind, sid], arow_v, sem)
            wcpy = pltpu.async_copy(wadj_hbm.at[ind, sid], wrow_v, sem)

            loc = ind - base

            @pl.when((loc >= 0) & (loc < CHUNK))
            def _owner():
                osl = pl.ds((loc >> 4) << 4, L)
                hit = lane == (loc & 15)
                open_v[osl] = jnp.where(hit, 0.0, open_v[osl])
                hist_v[osl] = jnp.where(hit, 1.0, hist_v[osl])

            acpy.wait()
            wcpy.wait()

            def upd_body(c, _):
                sl = pl.ds(c * L, L)
                gi = base + c * L + lane
                at_ind = gi == ind
                ai = plsc.bitcast(arow_v[sl], jnp.int32)
                ai = (ai + 0x7FFF + ((ai >> 16) & 1)) & jnp.int32(-65536)
                ar = plsc.bitcast(ai, jnp.float32)
                ar = jnp.where(at_ind, 0.0, ar)
                wr = wrow_v[sl]
                wr = jnp.where(wr == INF, 0.0, wr)
                wr = jnp.where(at_ind, 0.0, wr)
                ov = open_v[sl]
                hv = hist_v[sl]
                gv = g_v[sl]
                pv = par_v[sl]
                no = (1.0 - ov) * (1.0 - hv)
                nn = ar * no
                g2 = gsel + wr
                gt = (gv > g2).astype(jnp.float32)
                idxv = (no + ov * gt) * nn
                one_m = 1.0 - idxv
                g_v[sl] = g2 * idxv + gv * one_m
                open_v[sl] = jnp.clip(ov + idxv, 0.0, 1.0)
                par_v[sl] = ind_f * idxv + pv * one_m
                return 0

            lax.fori_loop(0, NCH, upd_body, 0, unroll=False)
            return 0

        lax.fori_loop(0, TMAX, step, 0, unroll=False)

        pltpu.sync_copy(hist_v, hist_out.at[sid])
        pltpu.sync_copy(par_v, par_sh.at[pl.ds(base, CHUNK)])
        plsc.subcore_barrier()

        @pl.when(sid == 0)
        def _backtrack():
            pltpu.sync_copy(par_sh, parall_v)

            def zero_body(c, _):
                path_v[pl.ds(c * L, L)] = jnp.zeros((L,), jnp.int32)
                return 0

            lax.fori_loop(0, N // L, zero_body, 0, unroll=False)

            def path_set(loc):
                locv = lax.broadcast(loc, (L,))
                plsc.store_scatter(path_v, [locv], jnp.ones((L,), jnp.int32),
                                   mask=lane == 0)
                pv = plsc.load_gather(parall_v, [locv])
                pvm = jnp.max(pv)
                i = pvm.astype(jnp.int32)
                return i - (i.astype(jnp.float32) > pvm).astype(jnp.int32)

            loc0 = path_set(goal)

            def chase(i, loc):
                return path_set(loc)

            lax.fori_loop(0, TMAX - 1, chase, loc0, unroll=False)
            pltpu.sync_copy(path_v, path_out)


@functools.partial(jax.jit, static_argnums=())
def _run(cost2, adj3, wadj3, sg):
    mesh = plsc.VectorSubcoreMesh(core_axis_name="c", subcore_axis_name="s")
    cp = pltpu.CompilerParams()
    if "needs_layout_passes" in pltpu.CompilerParams.__dataclass_fields__:
        cp = dataclasses.replace(cp, needs_layout_passes=False)
    f = pl.kernel(
        _astar_body,
        out_type=(
            jax.ShapeDtypeStruct((NS, CHUNK), jnp.float32),
            jax.ShapeDtypeStruct((N,), jnp.int32),
        ),
        mesh=mesh,
        scratch_types=[
            pltpu.VMEM((CHUNK,), jnp.float32),
            pltpu.VMEM((CHUNK,), jnp.float32),
            pltpu.VMEM((CHUNK,), jnp.float32),
            pltpu.VMEM((CHUNK,), jnp.float32),
            pltpu.VMEM((CHUNK,), jnp.float32),
            pltpu.VMEM((CHUNK,), jnp.float32),
            pltpu.VMEM((CHUNK,), jnp.float32),
            pltpu.VMEM((L,), jnp.int32),
            pltpu.VMEM((L,), jnp.float32),
            pltpu.VMEM((NS, L), jnp.float32),
            pltpu.VMEM((N,), jnp.float32),
            pltpu.VMEM((N,), jnp.int32),
            pltpu.VMEM_SHARED((NS, L), jnp.float32),
            pltpu.VMEM_SHARED((N,), jnp.float32),
            pltpu.SemaphoreType.DMA,
        ],
        compiler_params=cp,
    )
    return f(cost2, adj3, wadj3, sg)


def kernel(start_index, goal_index, cost_maps, nodes, adj, weighted_adj):
    del nodes
    sg = jnp.zeros((L,), jnp.int32)
    sg = sg.at[0].set(jnp.asarray(start_index, jnp.int32))
    sg = sg.at[1].set(jnp.asarray(goal_index, jnp.int32))
    cost2 = cost_maps.reshape(NS, CHUNK)
    adj3 = adj.reshape(N, NS, CHUNK)
    wadj3 = weighted_adj.reshape(N, NS, CHUNK)
    hist2, path1 = _run(cost2, adj3, wadj3, sg)
    return hist2.reshape(N), path1


# speedup vs baseline: 21.0730x; 1.0127x over previous
import dataclasses
import functools
import math

import jax
import jax.numpy as jnp
from jax import lax
from jax.experimental import pallas as pl
from jax.experimental.pallas import tpu as pltpu
from jax.experimental.pallas import tpu_sc as plsc

N = 4096
NS = 16
L = 16
CHUNK = N // NS
NCH = CHUNK // L
TMAX = int(0.05 * N)
INV_SQRT = 1.0 / math.sqrt(N)
INF = float("inf")


def _lane_iota():
    return lax.iota(jnp.int32, L)


def _astar_body(cost_hbm, adj_hbm, wadj_hbm, sg_hbm, hist_out, path_out,
                h_v, g_v, open_v, hist_v, par_v, arow_v, wrow_v, sg_v,
                row_v, cand_v, parall_v, path_v, cand_sh, par_sh, sem):
    cid = lax.axis_index("c")
    sid = lax.axis_index("s")

    @pl.when(cid == 0)
    def _work():
        base = sid * CHUNK
        lane = _lane_iota()

        pltpu.async_copy(sg_hbm, sg_v, sem).wait()
        sgv = sg_v[...]
        start = sgv[0]
        goal = sgv[1]
        goal_f = goal.astype(jnp.float32)

        pltpu.async_copy(cost_hbm.at[sid], h_v, sem).wait()
        pltpu.async_copy(wadj_hbm.at[start, sid], g_v, sem).wait()

        def init_body(c, _):
            sl = pl.ds(c * L, L)
            gi = base + c * L + lane
            gv = g_v[sl]
            gv = jnp.where(gv == INF, 0.0, gv)
            gv = jnp.where(gi == start, 0.0, gv)
            g_v[sl] = gv
            open_v[sl] = jnp.where(gi == start, 1.0, 0.0)
            hist_v[sl] = jnp.zeros((L,), jnp.float32)
            par_v[sl] = lax.broadcast(goal_f, (L,))
            return 0

        lax.fori_loop(0, NCH, init_body, 0, unroll=False)

        def step(t, _):
            best = jnp.float32(-1.0)
            bidx = jnp.int32(0)
            for c in range(NCH):
                sl = pl.ds(c * L, L)
                fe = jnp.exp((g_v[sl] * 0.5 + h_v[sl] * 0.5) * (-INV_SQRT))
                fe = fe * open_v[sl]
                m = jnp.max(fe)
                first = jnp.max(plsc.all_reduce_ffs(fe == m))
                take = m > best
                best = jnp.where(take, m, best)
                bidx = jnp.where(take, c * L + first, bidx)
            bv = g_v[pl.ds((bidx >> 4) << 4, L)]
            gbest = jnp.sum(jnp.where(lane == (bidx & 15), bv, 0.0))

            row = jnp.where(lane == 0, best, 0.0)
            row = jnp.where(lane == 1, (base + bidx).astype(jnp.float32), row)
            row = jnp.where(lane == 2, gbest, row)
            row_v[...] = row
            pltpu.sync_copy(row_v, cand_sh.at[sid])
            plsc.subcore_barrier()
            pltpu.sync_copy(cand_sh, cand_v)
            plsc.subcore_barrier()

            vals = plsc.load_gather(cand_v, [lane, jnp.zeros((L,), jnp.int32)])
            gm = jnp.max(vals)
            tstar = jnp.max(plsc.all_reduce_ffs(vals == gm))
            crow = cand_v[tstar, pl.ds(0, L)]
            ind = crow[1].astype(jnp.int32)
            gsel = crow[2]
            ind_f = ind.astype(jnp.float32)

            acpy = pltpu.async_copy(adj_hbm.at[ind, sid], arow_v, sem)
            wcpy = pltpu.async_copy(wadj_hbm.at[ind, sid], wrow_v, sem)

            loc = ind - base

            @pl.when((loc >= 0) & (loc < CHUNK))
            def _owner():
                osl = pl.ds((loc >> 4) << 4, L)
                hit = lane == (loc & 15)
                open_v[osl] = jnp.where(hit, 0.0, open_v[osl])
                hist_v[osl] = jnp.where(hit, 1.0, hist_v[osl])

            acpy.wait()
            wcpy.wait()

            for c in range(NCH):
                sl = pl.ds(c * L, L)
                gi = base + c * L + lane
                at_ind = gi == ind
                ai = plsc.bitcast(arow_v[sl], jnp.int32)
                ai = (ai + 0x7FFF + ((ai >> 16) & 1)) & jnp.int32(-65536)
                ar = plsc.bitcast(ai, jnp.float32)
                ar = jnp.where(at_ind, 0.0, ar)
                wr = wrow_v[sl]
                wr = jnp.where(wr == INF, 0.0, wr)
                wr = jnp.where(at_ind, 0.0, wr)
                ov = open_v[sl]
                hv = hist_v[sl]
                gv = g_v[sl]
                pv = par_v[sl]
                no = (1.0 - ov) * (1.0 - hv)
                nn = ar * no
                g2 = gsel + wr
                gt = (gv > g2).astype(jnp.float32)
                idxv = (no + ov * gt) * nn
                one_m = 1.0 - idxv
                g_v[sl] = g2 * idxv + gv * one_m
                open_v[sl] = jnp.clip(ov + idxv, 0.0, 1.0)
                par_v[sl] = ind_f * idxv + pv * one_m
            return 0

        lax.fori_loop(0, TMAX, step, 0, unroll=False)

        pltpu.sync_copy(hist_v, hist_out.at[sid])
        pltpu.sync_copy(par_v, par_sh.at[pl.ds(base, CHUNK)])
        plsc.subcore_barrier()

        @pl.when(sid == 0)
        def _backtrack():
            pltpu.sync_copy(par_sh, parall_v)

            def zero_body(c, _):
                path_v[pl.ds(c * L, L)] = jnp.zeros((L,), jnp.int32)
                return 0

            lax.fori_loop(0, N // L, zero_body, 0, unroll=False)

            def path_set(loc):
                locv = lax.broadcast(loc, (L,))
                plsc.store_scatter(path_v, [locv], jnp.ones((L,), jnp.int32),
                                   mask=lane == 0)
                pv = plsc.load_gather(parall_v, [locv])
                pvm = jnp.max(pv)
                i = pvm.astype(jnp.int32)
                return i - (i.astype(jnp.float32) > pvm).astype(jnp.int32)

            loc0 = path_set(goal)

            def chase(i, loc):
                return path_set(loc)

            lax.fori_loop(0, TMAX - 1, chase, loc0, unroll=False)
            pltpu.sync_copy(path_v, path_out)


@functools.partial(jax.jit, static_argnums=())
def _run(cost2, adj3, wadj3, sg):
    mesh = plsc.VectorSubcoreMesh(core_axis_name="c", subcore_axis_name="s")
    cp = pltpu.CompilerParams()
    if "needs_layout_passes" in pltpu.CompilerParams.__dataclass_fields__:
        cp = dataclasses.replace(cp, needs_layout_passes=False)
    f = pl.kernel(
        _astar_body,
        out_type=(
            jax.ShapeDtypeStruct((NS, CHUNK), jnp.float32),
            jax.ShapeDtypeStruct((N,), jnp.int32),
        ),
        mesh=mesh,
        scratch_types=[
            pltpu.VMEM((CHUNK,), jnp.float32),
            pltpu.VMEM((CHUNK,), jnp.float32),
            pltpu.VMEM((CHUNK,), jnp.float32),
            pltpu.VMEM((CHUNK,), jnp.float32),
            pltpu.VMEM((CHUNK,), jnp.float32),
            pltpu.VMEM((CHUNK,), jnp.float32),
            pltpu.VMEM((CHUNK,), jnp.float32),
            pltpu.VMEM((L,), jnp.int32),
            pltpu.VMEM((L,), jnp.float32),
            pltpu.VMEM((NS, L), jnp.float32),
            pltpu.VMEM((N,), jnp.float32),
            pltpu.VMEM((N,), jnp.int32),
            pltpu.VMEM_SHARED((NS, L), jnp.float32),
            pltpu.VMEM_SHARED((N,), jnp.float32),
            pltpu.SemaphoreType.DMA,
        ],
        compiler_params=cp,
    )
    return f(cost2, adj3, wadj3, sg)


def kernel(start_index, goal_index, cost_maps, nodes, adj, weighted_adj):
    del nodes
    sg = jnp.zeros((L,), jnp.int32)
    sg = sg.at[0].set(jnp.asarray(start_index, jnp.int32))
    sg = sg.at[1].set(jnp.asarray(goal_index, jnp.int32))
    cost2 = cost_maps.reshape(NS, CHUNK)
    adj3 = adj.reshape(N, NS, CHUNK)
    wadj3 = weighted_adj.reshape(N, NS, CHUNK)
    hist2, path1 = _run(cost2, adj3, wadj3, sg)
    return hist2.reshape(N), path1


# speedup vs baseline: 23.0550x; 1.0941x over previous
import dataclasses
import functools
import math

import jax
import jax.numpy as jnp
from jax import lax
from jax.experimental import pallas as pl
from jax.experimental.pallas import tpu as pltpu
from jax.experimental.pallas import tpu_sc as plsc

N = 4096
NS = 16
L = 16
CHUNK = N // NS
NCH = CHUNK // L
TMAX = int(0.05 * N)
INV_SQRT = 1.0 / math.sqrt(N)
INF = float("inf")


def _lane_iota():
    return lax.iota(jnp.int32, L)


def _astar_body(cost_hbm, adj_hbm, wadj_hbm, sg_hbm, hist_out, path_out,
                h_v, g_v, open_v, hist_v, par_v, arow_v, wrow_v, sg_v,
                row_v, cand_v, parall_v, path_v, cand_sh, par_sh, sem):
    cid = lax.axis_index("c")
    sid = lax.axis_index("s")

    @pl.when(cid == 0)
    def _work():
        base = sid * CHUNK
        lane = _lane_iota()

        pltpu.async_copy(sg_hbm, sg_v, sem).wait()
        sgv = sg_v[...]
        start = sgv[0]
        goal = sgv[1]
        goal_f = goal.astype(jnp.float32)

        pltpu.async_copy(cost_hbm.at[sid], h_v, sem).wait()
        pltpu.async_copy(wadj_hbm.at[start, sid], g_v, sem).wait()

        def init_body(c, _):
            sl = pl.ds(c * L, L)
            gi = base + c * L + lane
            gv = g_v[sl]
            gv = jnp.where(gv == INF, 0.0, gv)
            gv = jnp.where(gi == start, 0.0, gv)
            g_v[sl] = gv
            open_v[sl] = jnp.where(gi == start, 1.0, 0.0)
            hist_v[sl] = jnp.zeros((L,), jnp.float32)
            par_v[sl] = lax.broadcast(goal_f, (L,))
            return 0

        lax.fori_loop(0, NCH, init_body, 0, unroll=False)

        best0 = jnp.float32(-1.0)
        bidx0 = jnp.int32(0)
        for c in range(NCH):
            sl = pl.ds(c * L, L)
            fe = jnp.exp((g_v[sl] * 0.5 + h_v[sl] * 0.5) * (-INV_SQRT))
            fe = fe * open_v[sl]
            m = jnp.max(fe)
            first = jnp.max(plsc.all_reduce_ffs(fe == m))
            take = m > best0
            best0 = jnp.where(take, m, best0)
            bidx0 = jnp.where(take, c * L + first, bidx0)

        def step(t, carry):
            best, bidx = carry
            bv = g_v[pl.ds((bidx >> 4) << 4, L)]
            gbest = jnp.sum(jnp.where(lane == (bidx & 15), bv, 0.0))

            row = jnp.where(lane == 0, best, 0.0)
            row = jnp.where(lane == 1, (base + bidx).astype(jnp.float32), row)
            row = jnp.where(lane == 2, gbest, row)
            row_v[...] = row
            buf = t & 1
            pltpu.sync_copy(row_v, cand_sh.at[buf, sid])
            plsc.subcore_barrier()

            pltpu.sync_copy(cand_sh.at[buf], cand_v)

            vals = plsc.load_gather(cand_v, [lane, jnp.zeros((L,), jnp.int32)])
            gm = jnp.max(vals)
            tstar = jnp.max(plsc.all_reduce_ffs(vals == gm))
            crow = cand_v[tstar, pl.ds(0, L)]
            ind = crow[1].astype(jnp.int32)
            gsel = crow[2]
            ind_f = ind.astype(jnp.float32)

            acpy = pltpu.async_copy(adj_hbm.at[ind, sid], arow_v, sem)
            wcpy = pltpu.async_copy(wadj_hbm.at[ind, sid], wrow_v, sem)

            loc = ind - base

            @pl.when((loc >= 0) & (loc < CHUNK))
            def _owner():
                osl = pl.ds((loc >> 4) << 4, L)
                hit = lane == (loc & 15)
                open_v[osl] = jnp.where(hit, 0.0, open_v[osl])
                hist_v[osl] = jnp.where(hit, 1.0, hist_v[osl])

            acpy.wait()
            wcpy.wait()

            nbest = jnp.float32(-1.0)
            nbidx = jnp.int32(0)
            for c in range(NCH):
                sl = pl.ds(c * L, L)
                gi = base + c * L + lane
                at_ind = gi == ind
                ai = plsc.bitcast(arow_v[sl], jnp.int32)
                ai = (ai + 0x7FFF + ((ai >> 16) & 1)) & jnp.int32(-65536)
                ar = plsc.bitcast(ai, jnp.float32)
                ar = jnp.where(at_ind, 0.0, ar)
                wr = wrow_v[sl]
                wr = jnp.where(wr == INF, 0.0, wr)
                wr = jnp.where(at_ind, 0.0, wr)
                ov = open_v[sl]
                hv = hist_v[sl]
                gv = g_v[sl]
                pv = par_v[sl]
                no = (1.0 - ov) * (1.0 - hv)
                nn = ar * no
                g2 = gsel + wr
                gt = (gv > g2).astype(jnp.float32)
                idxv = (no + ov * gt) * nn
                one_m = 1.0 - idxv
                gnew = g2 * idxv + gv * one_m
                onew = jnp.clip(ov + idxv, 0.0, 1.0)
                g_v[sl] = gnew
                open_v[sl] = onew
                par_v[sl] = ind_f * idxv + pv * one_m
                fe = jnp.exp((gnew * 0.5 + h_v[sl] * 0.5) * (-INV_SQRT))
                fe = fe * onew
                m = jnp.max(fe)
                first = jnp.max(plsc.all_reduce_ffs(fe == m))
                take = m > nbest
                nbest = jnp.where(take, m, nbest)
                nbidx = jnp.where(take, c * L + first, nbidx)
            return nbest, nbidx

        lax.fori_loop(0, TMAX, step, (best0, bidx0), unroll=False)

        pltpu.sync_copy(hist_v, hist_out.at[sid])
        pltpu.sync_copy(par_v, par_sh.at[pl.ds(base, CHUNK)])
        plsc.subcore_barrier()

        @pl.when(sid == 0)
        def _backtrack():
            pltpu.sync_copy(par_sh, parall_v)

            def zero_body(c, _):
                path_v[pl.ds(c * L, L)] = jnp.zeros((L,), jnp.int32)
                return 0

            lax.fori_loop(0, N // L, zero_body, 0, unroll=False)

            def path_set(loc):
                locv = lax.broadcast(loc, (L,))
                plsc.store_scatter(path_v, [locv], jnp.ones((L,), jnp.int32),
                                   mask=lane == 0)
                pv = plsc.load_gather(parall_v, [locv])
                pvm = jnp.max(pv)
                i = pvm.astype(jnp.int32)
                return i - (i.astype(jnp.float32) > pvm).astype(jnp.int32)

            loc0 = path_set(goal)

            def chase(i, loc):
                return path_set(loc)

            lax.fori_loop(0, TMAX - 1, chase, loc0, unroll=False)
            pltpu.sync_copy(path_v, path_out)


@functools.partial(jax.jit, static_argnums=())
def _run(cost2, adj3, wadj3, sg):
    mesh = plsc.VectorSubcoreMesh(core_axis_name="c", subcore_axis_name="s")
    cp = pltpu.CompilerParams()
    if "needs_layout_passes" in pltpu.CompilerParams.__dataclass_fields__:
        cp = dataclasses.replace(cp, needs_layout_passes=False)
    f = pl.kernel(
        _astar_body,
        out_type=(
            jax.ShapeDtypeStruct((NS, CHUNK), jnp.float32),
            jax.ShapeDtypeStruct((N,), jnp.int32),
        ),
        mesh=mesh,
        scratch_types=[
            pltpu.VMEM((CHUNK,), jnp.float32),
            pltpu.VMEM((CHUNK,), jnp.float32),
            pltpu.VMEM((CHUNK,), jnp.float32),
            pltpu.VMEM((CHUNK,), jnp.float32),
            pltpu.VMEM((CHUNK,), jnp.float32),
            pltpu.VMEM((CHUNK,), jnp.float32),
            pltpu.VMEM((CHUNK,), jnp.float32),
            pltpu.VMEM((L,), jnp.int32),
            pltpu.VMEM((L,), jnp.float32),
            pltpu.VMEM((NS, L), jnp.float32),
            pltpu.VMEM((N,), jnp.float32),
            pltpu.VMEM((N,), jnp.int32),
            pltpu.VMEM_SHARED((2, NS, L), jnp.float32),
            pltpu.VMEM_SHARED((N,), jnp.float32),
            pltpu.SemaphoreType.DMA,
        ],
        compiler_params=cp,
    )
    return f(cost2, adj3, wadj3, sg)


def kernel(start_index, goal_index, cost_maps, nodes, adj, weighted_adj):
    del nodes
    sg = jnp.zeros((L,), jnp.int32)
    sg = sg.at[0].set(jnp.asarray(start_index, jnp.int32))
    sg = sg.at[1].set(jnp.asarray(goal_index, jnp.int32))
    cost2 = cost_maps.reshape(NS, CHUNK)
    adj3 = adj.reshape(N, NS, CHUNK)
    wadj3 = weighted_adj.reshape(N, NS, CHUNK)
    hist2, path1 = _run(cost2, adj3, wadj3, sg)
    return hist2.reshape(N), path1


# speedup vs baseline: 26.7583x; 1.1606x over previous
import dataclasses
import functools
import math

import jax
import jax.numpy as jnp
from jax import lax
from jax.experimental import pallas as pl
from jax.experimental.pallas import tpu as pltpu
from jax.experimental.pallas import tpu_sc as plsc

N = 4096
NS = 16
L = 16
CHUNK = N // NS
NCH = CHUNK // L
TMAX = int(0.05 * N)
INV_SQRT = 1.0 / math.sqrt(N)
INF = float("inf")


def _lane_iota():
    return lax.iota(jnp.int32, L)


def _astar_body(cost_hbm, adj_hbm, wadj_hbm, sg_hbm, hist_out, path_out,
                h_v, g_v, open_v, hist_v, par_v, arow_v, wrow_v,
                arow2_v, wrow2_v, sg_v,
                row_v, cand_v, parall_v, path_v, cand_sh, par_sh, sem, sem2):
    cid = lax.axis_index("c")
    sid = lax.axis_index("s")

    @pl.when(cid == 0)
    def _work():
        base = sid * CHUNK
        lane = _lane_iota()

        pltpu.async_copy(sg_hbm, sg_v, sem).wait()
        sgv = sg_v[...]
        start = sgv[0]
        goal = sgv[1]
        goal_f = goal.astype(jnp.float32)

        pltpu.async_copy(cost_hbm.at[sid], h_v, sem).wait()
        pltpu.async_copy(wadj_hbm.at[start, sid], g_v, sem).wait()

        def init_body(c, _):
            sl = pl.ds(c * L, L)
            gi = base + c * L + lane
            gv = g_v[sl]
            gv = jnp.where(gv == INF, 0.0, gv)
            gv = jnp.where(gi == start, 0.0, gv)
            g_v[sl] = gv
            open_v[sl] = jnp.where(gi == start, 1.0, 0.0)
            hist_v[sl] = jnp.zeros((L,), jnp.float32)
            par_v[sl] = lax.broadcast(goal_f, (L,))
            return 0

        lax.fori_loop(0, NCH, init_body, 0, unroll=False)

        best0 = jnp.float32(-1.0)
        bidx0 = jnp.int32(0)
        for c in range(NCH):
            sl = pl.ds(c * L, L)
            fe = jnp.exp((g_v[sl] * 0.5 + h_v[sl] * 0.5) * (-INV_SQRT))
            fe = fe * open_v[sl]
            m = jnp.max(fe)
            first = jnp.max(plsc.all_reduce_ffs(fe == m))
            take = m > best0
            best0 = jnp.where(take, m, best0)
            bidx0 = jnp.where(take, c * L + first, bidx0)

        pltpu.async_copy(adj_hbm.at[0, sid], arow2_v, sem2)
        pltpu.async_copy(wadj_hbm.at[0, sid], wrow2_v, sem2)

        def step(t, carry):
            best, bidx, spec, tb = carry
            bv = g_v[pl.ds((bidx >> 4) << 4, L)]
            gbest = jnp.sum(jnp.where(lane == (bidx & 15), bv, 0.0))

            row = jnp.where(lane == 0, best, 0.0)
            row = jnp.where(lane == 1, (base + bidx).astype(jnp.float32), row)
            row = jnp.where(lane == 2, gbest, row)
            row_v[...] = row
            buf = t & 1
            pltpu.sync_copy(row_v, cand_sh.at[buf, sid])
            plsc.subcore_barrier()

            pltpu.sync_copy(cand_sh.at[buf], cand_v)

            vals = plsc.load_gather(cand_v, [lane, jnp.zeros((L,), jnp.int32)])
            gm = jnp.max(vals)
            tstar = jnp.max(plsc.all_reduce_ffs(vals == gm))
            crow = cand_v[tstar, pl.ds(0, L)]
            ind = crow[1].astype(jnp.int32)
            gsel = crow[2]
            ind_f = ind.astype(jnp.float32)

            vals2 = jnp.where(lane == tstar, -1.0, vals)
            gm2 = jnp.max(vals2)
            t2 = jnp.max(plsc.all_reduce_ffs(vals2 == gm2))
            ind2 = cand_v[t2, pl.ds(0, L)][1].astype(jnp.int32)

            pltpu.make_async_copy(adj_hbm.at[0, sid], arow2_v, sem2).wait()
            pltpu.make_async_copy(wadj_hbm.at[0, sid], wrow2_v, sem2).wait()

            miss = ind != spec

            @pl.when(miss & (tb == 0))
            def _fetch0():
                pltpu.async_copy(adj_hbm.at[ind, sid], arow_v, sem)
                pltpu.async_copy(wadj_hbm.at[ind, sid], wrow_v, sem)

            @pl.when(miss & (tb == 1))
            def _fetch1():
                pltpu.async_copy(adj_hbm.at[ind, sid], arow2_v, sem)
                pltpu.async_copy(wadj_hbm.at[ind, sid], wrow2_v, sem)

            @pl.when(tb == 0)
            def _spec1():
                pltpu.async_copy(adj_hbm.at[ind2, sid], arow2_v, sem2)
                pltpu.async_copy(wadj_hbm.at[ind2, sid], wrow2_v, sem2)

            @pl.when(tb == 1)
            def _spec0():
                pltpu.async_copy(adj_hbm.at[ind2, sid], arow_v, sem2)
                pltpu.async_copy(wadj_hbm.at[ind2, sid], wrow_v, sem2)

            loc = ind - base

            @pl.when((loc >= 0) & (loc < CHUNK))
            def _owner():
                osl = pl.ds((loc >> 4) << 4, L)
                hit = lane == (loc & 15)
                open_v[osl] = jnp.where(hit, 0.0, open_v[osl])
                hist_v[osl] = jnp.where(hit, 1.0, hist_v[osl])

            @pl.when(miss)
            def _wait_fetch():
                pltpu.make_async_copy(adj_hbm.at[0, sid], arow_v, sem).wait()
                pltpu.make_async_copy(wadj_hbm.at[0, sid], wrow_v, sem).wait()

            use1 = tb == 1

            nbest = jnp.float32(-1.0)
            nbidx = jnp.int32(0)
            for c in range(NCH):
                sl = pl.ds(c * L, L)
                gi = base + c * L + lane
                at_ind = gi == ind
                ai = plsc.bitcast(jnp.where(use1, arow2_v[sl], arow_v[sl]),
                                  jnp.int32)
                ai = (ai + 0x7FFF + ((ai >> 16) & 1)) & jnp.int32(-65536)
                ar = plsc.bitcast(ai, jnp.float32)
                ar = jnp.where(at_ind, 0.0, ar)
                wr = jnp.where(use1, wrow2_v[sl], wrow_v[sl])
                wr = jnp.where(wr == INF, 0.0, wr)
                wr = jnp.where(at_ind, 0.0, wr)
                ov = open_v[sl]
                hv = hist_v[sl]
                gv = g_v[sl]
                pv = par_v[sl]
                no = (1.0 - ov) * (1.0 - hv)
                nn = ar * no
                g2 = gsel + wr
                gt = (gv > g2).astype(jnp.float32)
                idxv = (no + ov * gt) * nn
                one_m = 1.0 - idxv
                gnew = g2 * idxv + gv * one_m
                onew = jnp.clip(ov + idxv, 0.0, 1.0)
                g_v[sl] = gnew
                open_v[sl] = onew
                par_v[sl] = ind_f * idxv + pv * one_m
                fe = jnp.exp((gnew * 0.5 + h_v[sl] * 0.5) * (-INV_SQRT))
                fe = fe * onew
                m = jnp.max(fe)
                first = jnp.max(plsc.all_reduce_ffs(fe == m))
                take = m > nbest
                nbest = jnp.where(take, m, nbest)
                nbidx = jnp.where(take, c * L + first, nbidx)
            return nbest, nbidx, ind2, 1 - tb

        lax.fori_loop(0, TMAX, step,
                      (best0, bidx0, jnp.int32(-1), jnp.int32(1)),
                      unroll=False)

        pltpu.make_async_copy(adj_hbm.at[0, sid], arow2_v, sem2).wait()
        pltpu.make_async_copy(wadj_hbm.at[0, sid], wrow2_v, sem2).wait()

        pltpu.sync_copy(hist_v, hist_out.at[sid])
        pltpu.sync_copy(par_v, par_sh.at[pl.ds(base, CHUNK)])
        plsc.subcore_barrier()

        @pl.when(sid == 0)
        def _backtrack():
            pltpu.sync_copy(par_sh, parall_v)

            def zero_body(c, _):
                path_v[pl.ds(c * L, L)] = jnp.zeros((L,), jnp.int32)
                return 0

            lax.fori_loop(0, N // L, zero_body, 0, unroll=False)

            def path_set(loc):
                locv = lax.broadcast(loc, (L,))
                plsc.store_scatter(path_v, [locv], jnp.ones((L,), jnp.int32),
                                   mask=lane == 0)
                pv = plsc.load_gather(parall_v, [locv])
                pvm = jnp.max(pv)
                i = pvm.astype(jnp.int32)
                return i - (i.astype(jnp.float32) > pvm).astype(jnp.int32)

            loc0 = path_set(goal)

            def chase(i, loc):
                return path_set(loc)

            lax.fori_loop(0, TMAX - 1, chase, loc0, unroll=False)
            pltpu.sync_copy(path_v, path_out)


@functools.partial(jax.jit, static_argnums=())
def _run(cost2, adj3, wadj3, sg):
    mesh = plsc.VectorSubcoreMesh(core_axis_name="c", subcore_axis_name="s")
    cp = pltpu.CompilerParams()
    if "needs_layout_passes" in pltpu.CompilerParams.__dataclass_fields__:
        cp = dataclasses.replace(cp, needs_layout_passes=False)
    f = pl.kernel(
        _astar_body,
        out_type=(
            jax.ShapeDtypeStruct((NS, CHUNK), jnp.float32),
            jax.ShapeDtypeStruct((N,), jnp.int32),
        ),
        mesh=mesh,
        scratch_types=[
            pltpu.VMEM((CHUNK,), jnp.float32),
            pltpu.VMEM((CHUNK,), jnp.float32),
            pltpu.VMEM((CHUNK,), jnp.float32),
            pltpu.VMEM((CHUNK,), jnp.float32),
            pltpu.VMEM((CHUNK,), jnp.float32),
            pltpu.VMEM((CHUNK,), jnp.float32),
            pltpu.VMEM((CHUNK,), jnp.float32),
            pltpu.VMEM((CHUNK,), jnp.float32),
            pltpu.VMEM((CHUNK,), jnp.float32),
            pltpu.VMEM((L,), jnp.int32),
            pltpu.VMEM((L,), jnp.float32),
            pltpu.VMEM((NS, L), jnp.float32),
            pltpu.VMEM((N,), jnp.float32),
            pltpu.VMEM((N,), jnp.int32),
            pltpu.VMEM_SHARED((2, NS, L), jnp.float32),
            pltpu.VMEM_SHARED((N,), jnp.float32),
            pltpu.SemaphoreType.DMA,
            pltpu.SemaphoreType.DMA,
        ],
        compiler_params=cp,
    )
    return f(cost2, adj3, wadj3, sg)


def kernel(start_index, goal_index, cost_maps, nodes, adj, weighted_adj):
    del nodes
    sg = jnp.zeros((L,), jnp.int32)
    sg = sg.at[0].set(jnp.asarray(start_index, jnp.int32))
    sg = sg.at[1].set(jnp.asarray(goal_index, jnp.int32))
    cost2 = cost_maps.reshape(NS, CHUNK)
    adj3 = adj.reshape(N, NS, CHUNK)
    wadj3 = weighted_adj.reshape(N, NS, CHUNK)
    hist2, path1 = _run(cost2, adj3, wadj3, sg)
    return hist2.reshape(N), path1


# speedup vs baseline: 26.9496x; 1.0071x over previous
import dataclasses
import functools
import math

import jax
import jax.numpy as jnp
from jax import lax
from jax.experimental import pallas as pl
from jax.experimental.pallas import tpu as pltpu
from jax.experimental.pallas import tpu_sc as plsc

N = 4096
NS = 16
L = 16
CHUNK = N // NS
NCH = CHUNK // L
TMAX = int(0.05 * N)
INV_SQRT = 1.0 / math.sqrt(N)
INF = float("inf")


def _lane_iota():
    return lax.iota(jnp.int32, L)


def _astar_body(cost_hbm, adj_hbm, wadj_hbm, sg_hbm, hist_out, path_out,
                h_v, g_v, open_v, hist_v, par_v, arow_v, wrow_v,
                arow2_v, wrow2_v, sg_v,
                row_v, cand_v, parall_v, path_v, cand_sh, par_sh, sem, sem2):
    cid = lax.axis_index("c")
    sid = lax.axis_index("s")

    @pl.when(cid == 0)
    def _work():
        base = sid * CHUNK
        lane = _lane_iota()

        pltpu.async_copy(sg_hbm, sg_v, sem).wait()
        sgv = sg_v[...]
        start = sgv[0]
        goal = sgv[1]
        goal_f = goal.astype(jnp.float32)

        pltpu.async_copy(cost_hbm.at[sid], h_v, sem).wait()
        pltpu.async_copy(wadj_hbm.at[start, sid], g_v, sem).wait()

        def init_body(c, _):
            sl = pl.ds(c * L, L)
            gi = base + c * L + lane
            gv = g_v[sl]
            gv = jnp.where(gv == INF, 0.0, gv)
            gv = jnp.where(gi == start, 0.0, gv)
            g_v[sl] = gv
            open_v[sl] = jnp.where(gi == start, 1.0, 0.0)
            hist_v[sl] = jnp.zeros((L,), jnp.float32)
            par_v[sl] = lax.broadcast(goal_f, (L,))
            return 0

        lax.fori_loop(0, NCH, init_body, 0, unroll=False)

        vmax0 = lax.broadcast(jnp.float32(-1.0), (L,))
        cidx0 = jnp.zeros((L,), jnp.int32)
        for c in range(NCH):
            sl = pl.ds(c * L, L)
            fe = jnp.exp((g_v[sl] * 0.5 + h_v[sl] * 0.5) * (-INV_SQRT))
            fe = fe * open_v[sl]
            take = fe > vmax0
            vmax0 = jnp.where(take, fe, vmax0)
            cidx0 = jnp.where(take, c, cidx0)
        best0 = jnp.max(vmax0)
        bidx0 = jnp.min(jnp.where(vmax0 == best0,
                                  cidx0 * L + lane, CHUNK))

        pltpu.async_copy(adj_hbm.at[0, sid], arow2_v, sem2)
        pltpu.async_copy(wadj_hbm.at[0, sid], wrow2_v, sem2)

        def step(t, carry):
            best, bidx, spec, tb = carry
            bv = g_v[pl.ds((bidx >> 4) << 4, L)]
            gbest = jnp.sum(jnp.where(lane == (bidx & 15), bv, 0.0))

            row = jnp.where(lane == 0, best, 0.0)
            row = jnp.where(lane == 1, (base + bidx).astype(jnp.float32), row)
            row = jnp.where(lane == 2, gbest, row)
            row_v[...] = row
            buf = t & 1
            pltpu.sync_copy(row_v, cand_sh.at[buf, sid])
            plsc.subcore_barrier()

            pltpu.sync_copy(cand_sh.at[buf], cand_v)

            vals = plsc.load_gather(cand_v, [lane, jnp.zeros((L,), jnp.int32)])
            gm = jnp.max(vals)
            tstar = jnp.max(plsc.all_reduce_ffs(vals == gm))
            crow = cand_v[tstar, pl.ds(0, L)]
            ind = crow[1].astype(jnp.int32)
            gsel = crow[2]
            ind_f = ind.astype(jnp.float32)

            vals2 = jnp.where(lane == tstar, -1.0, vals)
            gm2 = jnp.max(vals2)
            t2 = jnp.max(plsc.all_reduce_ffs(vals2 == gm2))
            ind2 = cand_v[t2, pl.ds(0, L)][1].astype(jnp.int32)

            pltpu.make_async_copy(adj_hbm.at[0, sid], arow2_v, sem2).wait()
            pltpu.make_async_copy(wadj_hbm.at[0, sid], wrow2_v, sem2).wait()

            miss = ind != spec

            @pl.when(miss & (tb == 0))
            def _fetch0():
                pltpu.async_copy(adj_hbm.at[ind, sid], arow_v, sem)
                pltpu.async_copy(wadj_hbm.at[ind, sid], wrow_v, sem)

            @pl.when(miss & (tb == 1))
            def _fetch1():
                pltpu.async_copy(adj_hbm.at[ind, sid], arow2_v, sem)
                pltpu.async_copy(wadj_hbm.at[ind, sid], wrow2_v, sem)

            @pl.when(tb == 0)
            def _spec1():
                pltpu.async_copy(adj_hbm.at[ind2, sid], arow2_v, sem2)
                pltpu.async_copy(wadj_hbm.at[ind2, sid], wrow2_v, sem2)

            @pl.when(tb == 1)
            def _spec0():
                pltpu.async_copy(adj_hbm.at[ind2, sid], arow_v, sem2)
                pltpu.async_copy(wadj_hbm.at[ind2, sid], wrow_v, sem2)

            loc = ind - base

            @pl.when((loc >= 0) & (loc < CHUNK))
            def _owner():
                osl = pl.ds((loc >> 4) << 4, L)
                hit = lane == (loc & 15)
                open_v[osl] = jnp.where(hit, 0.0, open_v[osl])
                hist_v[osl] = jnp.where(hit, 1.0, hist_v[osl])

            @pl.when(miss)
            def _wait_fetch():
                pltpu.make_async_copy(adj_hbm.at[0, sid], arow_v, sem).wait()
                pltpu.make_async_copy(wadj_hbm.at[0, sid], wrow_v, sem).wait()

            use1 = tb == 1

            nvmax = lax.broadcast(jnp.float32(-1.0), (L,))
            ncidx = jnp.zeros((L,), jnp.int32)
            for c in range(NCH):
                sl = pl.ds(c * L, L)
                gi = base + c * L + lane
                at_ind = gi == ind
                ai = plsc.bitcast(jnp.where(use1, arow2_v[sl], arow_v[sl]),
                                  jnp.int32)
                ai = (ai + 0x7FFF + ((ai >> 16) & 1)) & jnp.int32(-65536)
                ar = plsc.bitcast(ai, jnp.float32)
                ar = jnp.where(at_ind, 0.0, ar)
                wr = jnp.where(use1, wrow2_v[sl], wrow_v[sl])
                wr = jnp.where(wr == INF, 0.0, wr)
                wr = jnp.where(at_ind, 0.0, wr)
                ov = open_v[sl]
                hv = hist_v[sl]
                gv = g_v[sl]
                pv = par_v[sl]
                no = (1.0 - ov) * (1.0 - hv)
                nn = ar * no
                g2 = gsel + wr
                gt = (gv > g2).astype(jnp.float32)
                idxv = (no + ov * gt) * nn
                one_m = 1.0 - idxv
                gnew = g2 * idxv + gv * one_m
                onew = jnp.clip(ov + idxv, 0.0, 1.0)
                g_v[sl] = gnew
                open_v[sl] = onew
                par_v[sl] = ind_f * idxv + pv * one_m
                fe = jnp.exp((gnew * 0.5 + h_v[sl] * 0.5) * (-INV_SQRT))
                fe = fe * onew
                take = fe > nvmax
                nvmax = jnp.where(take, fe, nvmax)
                ncidx = jnp.where(take, c, ncidx)
            nbest = jnp.max(nvmax)
            nbidx = jnp.min(jnp.where(nvmax == nbest,
                                      ncidx * L + lane, CHUNK))
            return nbest, nbidx, ind2, 1 - tb

        lax.fori_loop(0, TMAX, step,
                      (best0, bidx0, jnp.int32(-1), jnp.int32(1)),
                      unroll=False)

        pltpu.make_async_copy(adj_hbm.at[0, sid], arow2_v, sem2).wait()
        pltpu.make_async_copy(wadj_hbm.at[0, sid], wrow2_v, sem2).wait()

        pltpu.sync_copy(hist_v, hist_out.at[sid])
        pltpu.sync_copy(par_v, par_sh.at[pl.ds(base, CHUNK)])
        plsc.subcore_barrier()

        @pl.when(sid == 0)
        def _backtrack():
            pltpu.sync_copy(par_sh, parall_v)

            def zero_body(c, _):
                path_v[pl.ds(c * L, L)] = jnp.zeros((L,), jnp.int32)
                return 0

            lax.fori_loop(0, N // L, zero_body, 0, unroll=False)

            def path_set(loc):
                locv = lax.broadcast(loc, (L,))
                plsc.store_scatter(path_v, [locv], jnp.ones((L,), jnp.int32),
                                   mask=lane == 0)
                pv = plsc.load_gather(parall_v, [locv])
                pvm = jnp.max(pv)
                i = pvm.astype(jnp.int32)
                return i - (i.astype(jnp.float32) > pvm).astype(jnp.int32)

            loc0 = path_set(goal)

            def chase(i, loc):
                return path_set(loc)

            lax.fori_loop(0, TMAX - 1, chase, loc0, unroll=False)
            pltpu.sync_copy(path_v, path_out)


@functools.partial(jax.jit, static_argnums=())
def _run(cost2, adj3, wadj3, sg):
    mesh = plsc.VectorSubcoreMesh(core_axis_name="c", subcore_axis_name="s")
    cp = pltpu.CompilerParams()
    if "needs_layout_passes" in pltpu.CompilerParams.__dataclass_fields__:
        cp = dataclasses.replace(cp, needs_layout_passes=False)
    f = pl.kernel(
        _astar_body,
        out_type=(
            jax.ShapeDtypeStruct((NS, CHUNK), jnp.float32),
            jax.ShapeDtypeStruct((N,), jnp.int32),
        ),
        mesh=mesh,
        scratch_types=[
            pltpu.VMEM((CHUNK,), jnp.float32),
            pltpu.VMEM((CHUNK,), jnp.float32),
            pltpu.VMEM((CHUNK,), jnp.float32),
            pltpu.VMEM((CHUNK,), jnp.float32),
            pltpu.VMEM((CHUNK,), jnp.float32),
            pltpu.VMEM((CHUNK,), jnp.float32),
            pltpu.VMEM((CHUNK,), jnp.float32),
            pltpu.VMEM((CHUNK,), jnp.float32),
            pltpu.VMEM((CHUNK,), jnp.float32),
            pltpu.VMEM((L,), jnp.int32),
            pltpu.VMEM((L,), jnp.float32),
            pltpu.VMEM((NS, L), jnp.float32),
            pltpu.VMEM((N,), jnp.float32),
            pltpu.VMEM((N,), jnp.int32),
            pltpu.VMEM_SHARED((2, NS, L), jnp.float32),
            pltpu.VMEM_SHARED((N,), jnp.float32),
            pltpu.SemaphoreType.DMA,
            pltpu.SemaphoreType.DMA,
        ],
        compiler_params=cp,
    )
    return f(cost2, adj3, wadj3, sg)


def kernel(start_index, goal_index, cost_maps, nodes, adj, weighted_adj):
    del nodes
    sg = jnp.zeros((L,), jnp.int32)
    sg = sg.at[0].set(jnp.asarray(start_index, jnp.int32))
    sg = sg.at[1].set(jnp.asarray(goal_index, jnp.int32))
    cost2 = cost_maps.reshape(NS, CHUNK)
    adj3 = adj.reshape(N, NS, CHUNK)
    wadj3 = weighted_adj.reshape(N, NS, CHUNK)
    hist2, path1 = _run(cost2, adj3, wadj3, sg)
    return hist2.reshape(N), path1
